# baseline (device time: 316084 ns/iter reference)
import jax
import jax.numpy as jnp
from jax import lax
from jax.experimental import pallas as pl
from jax.experimental.pallas import tpu as pltpu

N_DEV = 4
M = 4096
N = 2048
CH = M // N_DEV
H = CH // 2
Q = H // 2


def _gelu(y):
    c = 0.7978845608028654
    return 0.5 * y * (1.0 + jnp.tanh(c * (y + 0.044715 * y * y * y)))


def kernel(x, w_mat):
    k_shard = x.shape[1]

    def body(x_hbm, w_ref, out_ref, xstage, pbuf,
             send_sems, recv_sems, load_sems):
        d = lax.axis_index("i")
        right = lax.rem(d + 1, N_DEV)
        left = lax.rem(d + N_DEV - 1, N_DEV)
        dev = (right, left)

        def rows(r, chunk, q):
            return out_ref.at[pl.ds(chunk * CH + r * H + q * Q, Q), :]

        def load_x(r, chunk):
            cp = pltpu.make_async_copy(
                x_hbm.at[pl.ds(chunk * CH + r * H, H), :],
                xstage.at[r], load_sems.at[r])
            cp.start()
            return cp

        def rdma(r, i, q, chunk):
            return pltpu.make_async_remote_copy(
                src_ref=rows(r, chunk, q), dst_ref=rows(r, chunk, q),
                send_sem=send_sems.at[r, i, q],
                recv_sem=recv_sems.at[r, i, q],
                device_id=(dev[r],), device_id_type=pl.DeviceIdType.MESH)

        loads = (load_x(0, d), load_x(1, d))

        barrier_sem = pltpu.get_barrier_semaphore()
        for nbr in (left, right):
            pl.semaphore_signal(barrier_sem, inc=1, device_id=(nbr,),
                                device_id_type=pl.DeviceIdType.MESH)
        pl.semaphore_wait(barrier_sem, 2)

        loads[0].wait()
        loads[1].wait()
        for q in range(2):
            for r in range(2):
                rows(r, d, q)[...] = jnp.dot(
                    xstage[r, q * Q:(q + 1) * Q], w_ref[...],
                    preferred_element_type=jnp.float32)
                rdma(r, 0, q, d).start()

        for s in range(N_DEV - 1):
            ca = lax.rem(d - s - 1 + 2 * N_DEV, N_DEV)
            cb = lax.rem(d + s + 1, N_DEV)
            loads = (load_x(0, ca), load_x(1, cb))
            for r in range(2):
                loads[r].wait()
                pbuf[r] = jnp.dot(xstage[r], w_ref[...],
                                  preferred_element_type=jnp.float32)

            for q in range(2):
                for r, rc in ((0, ca), (1, cb)):
                    rdma(r, s, q, rc).wait_recv()
                    acc = rows(r, rc, q)
                    acc[...] = acc[...] + pbuf[r, q * Q:(q + 1) * Q]
                    if s < N_DEV - 2:
                        rdma(r, s + 1, q, rc).start()
                    else:
                        acc[...] = _gelu(acc[...])
                        rdma(r, 3, q, rc).start()

        for s in range(N_DEV - 1):
            ra = lax.rem(d - s + 2 * N_DEV, N_DEV)
            rb = lax.rem(d + s, N_DEV)
            for q in range(2):
                for r, rc in ((0, ra), (1, rb)):
                    rdma(r, 3 + s, q, rc).wait_recv()
                    if s < N_DEV - 2:
                        rdma(r, 3 + s + 1, q, rc).start()

        for r in range(2):
            for i in range(6):
                for q in range(2):
                    rdma(r, i, q, d).wait_send()

    return pl.pallas_call(
        body,
        out_shape=jax.ShapeDtypeStruct((M, N), jnp.float32),
        in_specs=[
            pl.BlockSpec(memory_space=pl.ANY),
            pl.BlockSpec(memory_space=pltpu.VMEM),
        ],
        out_specs=pl.BlockSpec(memory_space=pltpu.VMEM),
        scratch_shapes=[
            pltpu.VMEM((2, H, k_shard), jnp.float32),
            pltpu.VMEM((2, H, N), jnp.float32),
            pltpu.SemaphoreType.DMA((2, 6, 2)),
            pltpu.SemaphoreType.DMA((2, 6, 2)),
            pltpu.SemaphoreType.DMA((2,)),
        ],
        compiler_params=pltpu.CompilerParams(
            collective_id=0,
            vmem_limit_bytes=60 * 1024 * 1024,
        ),
    )(x, w_mat)


# device time: 305237 ns/iter; 1.0355x vs baseline; 1.0355x over previous
import jax
import jax.numpy as jnp
from jax import lax
from jax.experimental import pallas as pl
from jax.experimental.pallas import tpu as pltpu

N_DEV = 4
M = 4096
N = 2048
CH = M // N_DEV
H = CH // 2
Q = H // 2


def _gelu(y):
    c = 0.7978845608028654
    return 0.5 * y * (1.0 + jnp.tanh(c * (y + 0.044715 * y * y * y)))


def kernel(x, w_mat):
    k_shard = x.shape[1]

    def body(x_hbm, w_ref, out_hbm, xstage, ibuf, pbuf, slots,
             send_sems, recv_sems, load_sems, store_sems):
        d = lax.axis_index("i")
        right = lax.rem(d + 1, N_DEV)
        left = lax.rem(d + N_DEV - 1, N_DEV)
        dev = (right, left)

        def row0(r, chunk):
            return chunk * CH + r * H

        def qsl(q):
            return pl.ds(q * Q, Q)

        def load_x(r, chunk):
            cp = pltpu.make_async_copy(
                x_hbm.at[pl.ds(row0(r, chunk), H), :],
                xstage.at[r], load_sems.at[r])
            cp.start()
            return cp

        def rs_rdma(r, s, q):
            src = ibuf.at[r, qsl(q)] if s == 0 else slots.at[r, s - 1, qsl(q)]
            return pltpu.make_async_remote_copy(
                src_ref=src, dst_ref=slots.at[r, s, qsl(q)],
                send_sem=send_sems.at[r, s, q], recv_sem=recv_sems.at[r, s, q],
                device_id=(dev[r],), device_id_type=pl.DeviceIdType.MESH)

        def ag_rdma(r, s, q, chunk, recv=False):
            dst = out_hbm.at[pl.ds(row0(r, chunk) + q * Q, Q), :]
            src = (slots.at[r, N_DEV - 2, qsl(q)] if (s == 0 and not recv)
                   else dst)
            return pltpu.make_async_remote_copy(
                src_ref=src, dst_ref=dst,
                send_sem=send_sems.at[r, 3 + s, q],
                recv_sem=recv_sems.at[r, 3 + s, q],
                device_id=(dev[r],), device_id_type=pl.DeviceIdType.MESH)

        loads = (load_x(0, d), load_x(1, d))

        barrier_sem = pltpu.get_barrier_semaphore()
        for nbr in (left, right):
            pl.semaphore_signal(barrier_sem, inc=1, device_id=(nbr,),
                                device_id_type=pl.DeviceIdType.MESH)
        pl.semaphore_wait(barrier_sem, 2)

        loads[0].wait()
        loads[1].wait()
        for q in range(2):
            for r in range(2):
                ibuf[r, qsl(q)] = jnp.dot(
                    xstage[r, qsl(q)], w_ref[...],
                    preferred_element_type=jnp.float32)
                rs_rdma(r, s=0, q=q).start()

        for s in range(N_DEV - 1):
            ca = lax.rem(d - s - 1 + 2 * N_DEV, N_DEV)
            cb = lax.rem(d + s + 1, N_DEV)
            loads = (load_x(0, ca), load_x(1, cb))
            for r in range(2):
                loads[r].wait()
                pbuf[r] = jnp.dot(xstage[r], w_ref[...],
                                  preferred_element_type=jnp.float32)

            for q in range(2):
                for r in range(2):
                    rs_rdma(r, s, q).wait_recv()
                    slots[r, s, qsl(q)] = slots[r, s, qsl(q)] + pbuf[r, qsl(q)]
                    if s < N_DEV - 2:
                        rs_rdma(r, s + 1, q).start()
                    else:
                        slots[r, s, qsl(q)] = _gelu(slots[r, s, qsl(q)])
                        own = lax.rem(d + (1 if r == 0 else N_DEV - 1), N_DEV)
                        pltpu.make_async_copy(
                            slots.at[r, s, qsl(q)],
                            out_hbm.at[pl.ds(row0(r, own) + q * Q, Q), :],
                            store_sems.at[r, q]).start()
                        ag_rdma(r, 0, q, own).start()

        for s in range(N_DEV - 1):
            ra = lax.rem(d - s + 2 * N_DEV, N_DEV)
            rb = lax.rem(d + s, N_DEV)
            for q in range(2):
                for r, rc in ((0, ra), (1, rb)):
                    ag_rdma(r, s, q, rc, recv=True).wait_recv()
                    if s < N_DEV - 2:
                        ag_rdma(r, s + 1, q, rc).start()

        for r in range(2):
            for q in range(2):
                pltpu.make_async_copy(
                    slots.at[r, N_DEV - 2, qsl(q)],
                    out_hbm.at[pl.ds(0, Q), :],
                    store_sems.at[r, q]).wait()

        for r in range(2):
            for i in range(6):
                for q in range(2):
                    pltpu.make_async_remote_copy(
                        src_ref=slots.at[r, 0, qsl(0)],
                        dst_ref=slots.at[r, 0, qsl(0)],
                        send_sem=send_sems.at[r, i, q],
                        recv_sem=recv_sems.at[r, i, q],
                        device_id=(dev[r],),
                        device_id_type=pl.DeviceIdType.MESH).wait_send()

    return pl.pallas_call(
        body,
        out_shape=jax.ShapeDtypeStruct((M, N), jnp.float32),
        in_specs=[
            pl.BlockSpec(memory_space=pl.ANY),
            pl.BlockSpec(memory_space=pltpu.VMEM),
        ],
        out_specs=pl.BlockSpec(memory_space=pl.ANY),
        scratch_shapes=[
            pltpu.VMEM((2, H, k_shard), jnp.float32),
            pltpu.VMEM((2, H, N), jnp.float32),
            pltpu.VMEM((2, H, N), jnp.float32),
            pltpu.VMEM((2, N_DEV - 1, H, N), jnp.float32),
            pltpu.SemaphoreType.DMA((2, 6, 2)),
            pltpu.SemaphoreType.DMA((2, 6, 2)),
            pltpu.SemaphoreType.DMA((2,)),
            pltpu.SemaphoreType.DMA((2, 2)),
        ],
        compiler_params=pltpu.CompilerParams(
            collective_id=0,
            vmem_limit_bytes=60 * 1024 * 1024,
        ),
    )(x, w_mat)


# device time: 304688 ns/iter; 1.0374x vs baseline; 1.0018x over previous
import jax
import jax.numpy as jnp
from jax import lax
from jax.experimental import pallas as pl
from jax.experimental.pallas import tpu as pltpu

N_DEV = 4
M = 4096
N = 2048
CH = M // N_DEV
H = CH // 2
NQ = 4
Q = H // NQ


def _gelu(y):
    c = 0.7978845608028654
    return 0.5 * y * (1.0 + jnp.tanh(c * (y + 0.044715 * y * y * y)))


def kernel(x, w_mat):
    k_shard = x.shape[1]

    def body(x_hbm, w_ref, out_hbm, xstage, ibuf, pbuf, slots,
             send_sems, recv_sems, load_sems, store_sems):
        d = lax.axis_index("i")
        right = lax.rem(d + 1, N_DEV)
        left = lax.rem(d + N_DEV - 1, N_DEV)
        dev = (right, left)

        def row0(r, chunk):
            return chunk * CH + r * H

        def qsl(q):
            return pl.ds(q * Q, Q)

        def load_x(r, chunk):
            cp = pltpu.make_async_copy(
                x_hbm.at[pl.ds(row0(r, chunk), H), :],
                xstage.at[r], load_sems.at[r])
            cp.start()
            return cp

        def rs_rdma(r, s, q):
            src = ibuf.at[r, qsl(q)] if s == 0 else slots.at[r, s - 1, qsl(q)]
            return pltpu.make_async_remote_copy(
                src_ref=src, dst_ref=slots.at[r, s, qsl(q)],
                send_sem=send_sems.at[r, s, q], recv_sem=recv_sems.at[r, s, q],
                device_id=(dev[r],), device_id_type=pl.DeviceIdType.MESH)

        def ag_rdma(r, s, q, chunk, recv=False):
            dst = out_hbm.at[pl.ds(row0(r, chunk) + q * Q, Q), :]
            src = (slots.at[r, N_DEV - 2, qsl(q)] if (s == 0 and not recv)
                   else dst)
            return pltpu.make_async_remote_copy(
                src_ref=src, dst_ref=dst,
                send_sem=send_sems.at[r, 3 + s, q],
                recv_sem=recv_sems.at[r, 3 + s, q],
                device_id=(dev[r],), device_id_type=pl.DeviceIdType.MESH)

        loads = (load_x(0, d), load_x(1, d))

        barrier_sem = pltpu.get_barrier_semaphore()
        for nbr in (left, right):
            pl.semaphore_signal(barrier_sem, inc=1, device_id=(nbr,),
                                device_id_type=pl.DeviceIdType.MESH)
        pl.semaphore_wait(barrier_sem, 2)

        loads[0].wait()
        loads[1].wait()
        for q in range(NQ):
            for r in range(2):
                ibuf[r, qsl(q)] = jnp.dot(
                    xstage[r, qsl(q)], w_ref[...],
                    preferred_element_type=jnp.float32)
                rs_rdma(r, s=0, q=q).start()

        for s in range(N_DEV - 1):
            ca = lax.rem(d - s - 1 + 2 * N_DEV, N_DEV)
            cb = lax.rem(d + s + 1, N_DEV)
            loads = (load_x(0, ca), load_x(1, cb))
            for r in range(2):
                loads[r].wait()
                pbuf[r] = jnp.dot(xstage[r], w_ref[...],
                                  preferred_element_type=jnp.float32)

            for q in range(NQ):
                for r in range(2):
                    rs_rdma(r, s, q).wait_recv()
                    slots[r, s, qsl(q)] = slots[r, s, qsl(q)] + pbuf[r, qsl(q)]
                    if s < N_DEV - 2:
                        rs_rdma(r, s + 1, q).start()
                    else:
                        slots[r, s, qsl(q)] = _gelu(slots[r, s, qsl(q)])
                        own = lax.rem(d + (1 if r == 0 else N_DEV - 1), N_DEV)
                        pltpu.make_async_copy(
                            slots.at[r, s, qsl(q)],
                            out_hbm.at[pl.ds(row0(r, own) + q * Q, Q), :],
                            store_sems.at[r, q]).start()
                        ag_rdma(r, 0, q, own).start()

        for s in range(N_DEV - 1):
            ra = lax.rem(d - s + 2 * N_DEV, N_DEV)
            rb = lax.rem(d + s, N_DEV)
            for q in range(NQ):
                for r, rc in ((0, ra), (1, rb)):
                    ag_rdma(r, s, q, rc, recv=True).wait_recv()
                    if s < N_DEV - 2:
                        ag_rdma(r, s + 1, q, rc).start()

        for r in range(2):
            for q in range(NQ):
                pltpu.make_async_copy(
                    slots.at[r, N_DEV - 2, qsl(q)],
                    out_hbm.at[pl.ds(0, Q), :],
                    store_sems.at[r, q]).wait()

        for r in range(2):
            for i in range(6):
                for q in range(NQ):
                    pltpu.make_async_remote_copy(
                        src_ref=slots.at[r, 0, qsl(0)],
                        dst_ref=slots.at[r, 0, qsl(0)],
                        send_sem=send_sems.at[r, i, q],
                        recv_sem=recv_sems.at[r, i, q],
                        device_id=(dev[r],),
                        device_id_type=pl.DeviceIdType.MESH).wait_send()

    return pl.pallas_call(
        body,
        out_shape=jax.ShapeDtypeStruct((M, N), jnp.float32),
        in_specs=[
            pl.BlockSpec(memory_space=pl.ANY),
            pl.BlockSpec(memory_space=pltpu.VMEM),
        ],
        out_specs=pl.BlockSpec(memory_space=pl.ANY),
        scratch_shapes=[
            pltpu.VMEM((2, H, k_shard), jnp.float32),
            pltpu.VMEM((2, H, N), jnp.float32),
            pltpu.VMEM((2, H, N), jnp.float32),
            pltpu.VMEM((2, N_DEV - 1, H, N), jnp.float32),
            pltpu.SemaphoreType.DMA((2, 6, NQ)),
            pltpu.SemaphoreType.DMA((2, 6, NQ)),
            pltpu.SemaphoreType.DMA((2,)),
            pltpu.SemaphoreType.DMA((2, NQ)),
        ],
        compiler_params=pltpu.CompilerParams(
            collective_id=0,
            vmem_limit_bytes=60 * 1024 * 1024,
        ),
    )(x, w_mat)
